# baseline (speedup 1.0000x reference)
import functools

import jax
import jax.numpy as jnp
from jax import lax
from jax.experimental import pallas as pl
from jax.experimental.pallas import tpu as pltpu
from jax.experimental.pallas import tpu_sc as plsc

N_NODES = 10000
DIM = 128
NC = 2
NS = 16
NW = NC * NS
CHUNK = 128
CPT = 80
NBUF = 4
E_PAD = NW * CPT * CHUNK
ACC_ROWS = 10240
BLK = 1024


def _sc_scatter_body(src_hbm, dst_hbm, w_hbm, out_hbm,
                     src_v, dst_v, rows_v, acc, gsem):
    c = lax.axis_index("c")
    s = lax.axis_index("s")
    wid = c * NS + s

    pltpu.sync_copy(src_hbm.at[wid], src_v)
    pltpu.sync_copy(dst_hbm.at[wid], dst_v)

    def _zrow(i, _):
        def _zcol(j, _):
            rows_v[0, i, pl.ds(j * 32, 32)] = jnp.zeros((32,), jnp.bfloat16)
            return 0
        return lax.fori_loop(0, DIM // 32, _zcol, 0)
    lax.fori_loop(0, CHUNK, _zrow, 0)

    rows_per_tile = ACC_ROWS // NS
    r0 = s * rows_per_tile
    for b in range(rows_per_tile // CHUNK):
        pltpu.sync_copy(rows_v.at[0], acc.at[pl.ds(r0 + b * CHUNK, CHUNK)])
    plsc.subcore_barrier()

    for b in range(NBUF):
        pltpu.async_copy(w_hbm.at[src_v.at[b]], rows_v.at[b], gsem.at[b])

    def _round(g, _):
        for b in range(NBUF):
            j = g * NBUF + b
            pltpu.make_async_copy(w_hbm.at[pl.ds(0, CHUNK)], rows_v.at[b],
                                  gsem.at[b]).wait()
            pltpu.sync_copy(rows_v.at[b], acc.at[dst_v.at[j]], add=True)

            @pl.when(j + NBUF < CPT)
            def _():
                pltpu.async_copy(w_hbm.at[src_v.at[j + NBUF]], rows_v.at[b],
                                 gsem.at[b])
        return 0
    lax.fori_loop(0, CPT // NBUF, _round, 0)
    plsc.subcore_barrier()

    for b in range(rows_per_tile // CHUNK):
        pltpu.sync_copy(acc.at[pl.ds(r0 + b * CHUNK, CHUNK)],
                        out_hbm.at[c].at[pl.ds(r0 + b * CHUNK, CHUNK)])


_sc_scatter = functools.partial(
    pl.kernel,
    out_type=jax.ShapeDtypeStruct((NC, ACC_ROWS, DIM), jnp.bfloat16),
    mesh=plsc.VectorSubcoreMesh(core_axis_name="c", subcore_axis_name="s"),
    scratch_types=[
        pltpu.VMEM((CPT, CHUNK), jnp.int32),
        pltpu.VMEM((CPT, CHUNK), jnp.int32),
        pltpu.VMEM((NBUF, CHUNK, DIM), jnp.bfloat16),
        pltpu.VMEM_SHARED((ACC_ROWS, DIM), jnp.bfloat16),
        pltpu.SemaphoreType.DMA((NBUF,)),
    ],
    compiler_params=pltpu.CompilerParams(use_tc_tiling_on_sc=False),
)(_sc_scatter_body)


def _dense_body(p0, p1, xr, be, wc1, bc1, wn, bn, wc2, bc2, wf, bf, yr):
    f32 = jnp.float32
    S = p0[0].astype(f32) + p1[0].astype(f32) + be[...]
    out = S + jnp.dot(S, wc1[...], preferred_element_type=f32) + bc1[...]
    xn = jnp.dot(xr[...], wn[...], preferred_element_type=f32) + bn[...]
    out = out + xn + jnp.dot(xn, wc2[...], preferred_element_type=f32) + bc2[...]
    yr[...] = jnp.dot(jnp.maximum(out, 0.0), wf[...], preferred_element_type=f32) + bf[...]


def _dense_call(partials, x_pad, b_edge, W_cat1, b_cat1, W_node, b_node,
                W_cat2, b_cat2, W_final, b_final):
    grid = ACC_ROWS // BLK
    row_spec = pl.BlockSpec((BLK, DIM), lambda i: (i, 0))
    w_spec = pl.BlockSpec((DIM, DIM), lambda i: (0, 0))
    b_spec = pl.BlockSpec((1, DIM), lambda i: (0, 0))
    return pl.pallas_call(
        _dense_body,
        grid=(grid,),
        in_specs=[
            pl.BlockSpec((1, BLK, DIM), lambda i: (0, i, 0)),
            pl.BlockSpec((1, BLK, DIM), lambda i: (1, i, 0)),
            row_spec, b_spec, w_spec, b_spec, w_spec, b_spec, w_spec, b_spec,
            w_spec, b_spec,
        ],
        out_specs=row_spec,
        out_shape=jax.ShapeDtypeStruct((ACC_ROWS, DIM), jnp.float32),
    )(partials, partials, x_pad, b_edge, W_cat1, b_cat1, W_node, b_node,
      W_cat2, b_cat2, W_final, b_final)


def kernel(x, edge_index, W_edge, b_edge, W_node, b_node, W_cat1, b_cat1,
           W_cat2, b_cat2, W_final, b_final):
    n, d = W_edge.shape
    e = edge_index.shape[1]
    pad = E_PAD - e
    src = jnp.concatenate([edge_index[0], jnp.zeros((pad,), edge_index.dtype)])
    dst = jnp.concatenate([edge_index[1],
                           jnp.full((pad,), N_NODES, edge_index.dtype)])
    src = src.reshape(NW, CPT, CHUNK)
    dst = dst.reshape(NW, CPT, CHUNK)

    partials = _sc_scatter(src, dst, W_edge.astype(jnp.bfloat16))

    x_pad = jnp.pad(x, ((0, ACC_ROWS - n), (0, 0)))
    y = _dense_call(partials, x_pad,
                    b_edge.reshape(1, d), W_cat1, b_cat1.reshape(1, d),
                    W_node, b_node.reshape(1, d), W_cat2, b_cat2.reshape(1, d),
                    W_final, b_final.reshape(1, d))
    return y[:n]


# speedup vs baseline: 1.1679x; 1.1679x over previous
import functools

import jax
import jax.numpy as jnp
from jax import lax
from jax.experimental import pallas as pl
from jax.experimental.pallas import tpu as pltpu
from jax.experimental.pallas import tpu_sc as plsc

N_NODES = 10000
DIM = 128
NC = 2
NS = 16
NW = NC * NS
CHUNK = 128
CPT = 79
E_PAD = NW * CPT * CHUNK
ACC_ROWS = 10240
BLK = 1024


def _sc_scatter_body(src_hbm, dst_hbm, w_hbm, out_hbm,
                     src_v, dst_v, rows_a, acc, gsem):
    c = lax.axis_index("c")
    s = lax.axis_index("s")
    wid = c * NS + s

    pltpu.sync_copy(src_hbm.at[wid], src_v)
    pltpu.sync_copy(dst_hbm.at[wid], dst_v)

    def _zrow(i, _):
        def _zcol(j, _):
            rows_a[i, pl.ds(j * 32, 32)] = jnp.zeros((32,), jnp.bfloat16)
            return 0
        return lax.fori_loop(0, DIM // 32, _zcol, 0)
    lax.fori_loop(0, CHUNK, _zrow, 0)

    rows_per_tile = ACC_ROWS // NS
    r0 = s * rows_per_tile
    for b in range(rows_per_tile // CHUNK):
        pltpu.sync_copy(rows_a, acc.at[pl.ds(r0 + b * CHUNK, CHUNK)])
    plsc.subcore_barrier()

    def _body(j, _):
        pltpu.async_copy(w_hbm.at[src_v.at[j]], rows_a, gsem).wait()
        pltpu.sync_copy(rows_a, acc.at[dst_v.at[j]], add=True)
        return 0
    lax.fori_loop(0, CPT, _body, 0)
    plsc.subcore_barrier()

    for b in range(rows_per_tile // CHUNK):
        pltpu.sync_copy(acc.at[pl.ds(r0 + b * CHUNK, CHUNK)],
                        out_hbm.at[c].at[pl.ds(r0 + b * CHUNK, CHUNK)])


_sc_scatter = functools.partial(
    pl.kernel,
    out_type=jax.ShapeDtypeStruct((NC, ACC_ROWS, DIM), jnp.bfloat16),
    mesh=plsc.VectorSubcoreMesh(core_axis_name="c", subcore_axis_name="s"),
    scratch_types=[
        pltpu.VMEM((CPT, CHUNK), jnp.int32),
        pltpu.VMEM((CPT, CHUNK), jnp.int32),
        pltpu.VMEM((CHUNK, DIM), jnp.bfloat16),
        pltpu.VMEM_SHARED((ACC_ROWS, DIM), jnp.bfloat16),
        pltpu.SemaphoreType.DMA,
    ],
    compiler_params=pltpu.CompilerParams(use_tc_tiling_on_sc=False),
)(_sc_scatter_body)


def _dense_body(p0, p1, xr, be, wc1, bc1, wn, bn, wc2, bc2, wf, bf, yr):
    f32 = jnp.float32
    S = p0[0].astype(f32) + p1[0].astype(f32) + be[...]
    out = S + jnp.dot(S, wc1[...], preferred_element_type=f32) + bc1[...]
    xn = jnp.dot(xr[...], wn[...], preferred_element_type=f32) + bn[...]
    out = out + xn + jnp.dot(xn, wc2[...], preferred_element_type=f32) + bc2[...]
    yr[...] = jnp.dot(jnp.maximum(out, 0.0), wf[...], preferred_element_type=f32) + bf[...]


def _dense_call(partials, x_pad, b_edge, W_cat1, b_cat1, W_node, b_node,
                W_cat2, b_cat2, W_final, b_final):
    grid = ACC_ROWS // BLK
    row_spec = pl.BlockSpec((BLK, DIM), lambda i: (i, 0))
    w_spec = pl.BlockSpec((DIM, DIM), lambda i: (0, 0))
    b_spec = pl.BlockSpec((1, DIM), lambda i: (0, 0))
    return pl.pallas_call(
        _dense_body,
        grid=(grid,),
        in_specs=[
            pl.BlockSpec((1, BLK, DIM), lambda i: (0, i, 0)),
            pl.BlockSpec((1, BLK, DIM), lambda i: (1, i, 0)),
            row_spec, b_spec, w_spec, b_spec, w_spec, b_spec, w_spec, b_spec,
            w_spec, b_spec,
        ],
        out_specs=row_spec,
        out_shape=jax.ShapeDtypeStruct((ACC_ROWS, DIM), jnp.float32),
    )(partials, partials, x_pad, b_edge, W_cat1, b_cat1, W_node, b_node,
      W_cat2, b_cat2, W_final, b_final)


def kernel(x, edge_index, W_edge, b_edge, W_node, b_node, W_cat1, b_cat1,
           W_cat2, b_cat2, W_final, b_final):
    n, d = W_edge.shape
    e = edge_index.shape[1]
    pad = E_PAD - e
    src = jnp.concatenate([edge_index[0], jnp.zeros((pad,), edge_index.dtype)])
    dst = jnp.concatenate([edge_index[1],
                           jnp.full((pad,), N_NODES, edge_index.dtype)])
    src = src.reshape(NW, CPT, CHUNK)
    dst = dst.reshape(NW, CPT, CHUNK)

    partials = _sc_scatter(src, dst, W_edge.astype(jnp.bfloat16))

    x_pad = jnp.pad(x, ((0, ACC_ROWS - n), (0, 0)))
    y = _dense_call(partials, x_pad,
                    b_edge.reshape(1, d), W_cat1, b_cat1.reshape(1, d),
                    W_node, b_node.reshape(1, d), W_cat2, b_cat2.reshape(1, d),
                    W_final, b_final.reshape(1, d))
    return y[:n]
